# Initial kernel scaffold; baseline (speedup 1.0000x reference)
import jax
import jax.numpy as jnp
from jax.experimental import pallas as pl


def kernel(feat, edge_index, W0, W1, W2, b2, L0, L1, L2, g0, be0, g1, be1):
    raise NotImplementedError("write your pallas kernel here")


# speedup vs baseline: 4.2410x; 4.2410x over previous
import functools

import jax
import jax.numpy as jnp
from jax import lax
from jax.experimental import pallas as pl
from jax.experimental.pallas import tpu as pltpu
from jax.experimental.pallas import tpu_sc as plsc

_N = 10000
_E = 320000
_F = 128
_LANES = 16
_NC = 2
_NS = 16
_NW = _NC * _NS

_EPT = _E // _NW
_K = 80
_NCHUNK = _EPT // _K
_NP = 10240
_RPT = _NP // _NS


def _sc_mesh():
    return plsc.VectorSubcoreMesh(core_axis_name="c", subcore_axis_name="s")


def _sc_degrees(eidx):
    ept = _E // _NS
    nchunk = ept // _K

    @functools.partial(
        pl.kernel,
        out_type=jax.ShapeDtypeStruct((_NC, _NP, _F), jnp.float32),
        mesh=_sc_mesh(),
        scratch_types=[
            pltpu.VMEM((_K,), jnp.int32),
            pltpu.VMEM((_K, _F), jnp.float32),
            pltpu.VMEM_SHARED((_NP, _F), jnp.float32),
        ],
    )
    def k(eidx_h, out_h, idx_v, ones_v, cnt_sh):
        c = lax.axis_index("c")
        s = lax.axis_index("s")

        def fill(val):
            def f(i, _):
                ones_v[i // (_F // _LANES),
                       pl.ds((i % (_F // _LANES)) * _LANES, _LANES)] = (
                           jnp.full((_LANES,), val, jnp.float32))
                return 0
            lax.fori_loop(0, _K * (_F // _LANES), f, 0)

        fill(0.0)
        for t in range(_RPT // _K):
            pltpu.sync_copy(ones_v,
                            cnt_sh.at[pl.ds(s * _RPT + t * _K, _K)])
        fill(1.0)
        plsc.subcore_barrier()

        def body(j, _):
            base = c * _E + s * ept + j * _K
            pltpu.sync_copy(eidx_h.at[pl.ds(base, _K)], idx_v)
            pltpu.sync_copy(ones_v, cnt_sh.at[idx_v], add=True)
            return 0

        lax.fori_loop(0, nchunk, body, 0)
        plsc.subcore_barrier()

        pltpu.sync_copy(cnt_sh.at[pl.ds(s * _RPT, _RPT)],
                        out_h.at[c, pl.ds(s * _RPT, _RPT)])

    return k(eidx)


def _sc_agg(p, src, dst):
    @functools.partial(
        pl.kernel,
        out_type=jax.ShapeDtypeStruct((_NC, _NP, _F), jnp.float32),
        mesh=_sc_mesh(),
        scratch_types=[
            pltpu.VMEM((_K,), jnp.int32),
            pltpu.VMEM((_K,), jnp.int32),
            pltpu.VMEM((_K, _F), jnp.float32),
            pltpu.VMEM_SHARED((_NP, _F), jnp.float32),
            pltpu.SemaphoreType.DMA,
        ],
    )
    def k(p_h, src_h, dst_h, out_h, sidx, didx, rows, agg_sh, sem):
        c = lax.axis_index("c")
        s = lax.axis_index("s")

        def zrow(i, _):
            rows[i // (_F // _LANES),
                 pl.ds((i % (_F // _LANES)) * _LANES, _LANES)] = (
                     jnp.zeros((_LANES,), jnp.float32))
            return 0

        lax.fori_loop(0, _K * (_F // _LANES), zrow, 0)
        for t in range(_RPT // _K):
            pltpu.sync_copy(rows,
                            agg_sh.at[pl.ds(s * _RPT + t * _K, _K)])
        plsc.subcore_barrier()

        def body(j, _):
            base = c * (_E // _NC) + s * _EPT + j * _K
            pltpu.sync_copy(src_h.at[pl.ds(base, _K)], sidx)
            cp = pltpu.async_copy(p_h.at[sidx], rows, sem)
            pltpu.sync_copy(dst_h.at[pl.ds(base, _K)], didx)
            cp.wait()
            pltpu.sync_copy(rows, agg_sh.at[didx], add=True)
            return 0

        lax.fori_loop(0, _NCHUNK, body, 0)
        plsc.subcore_barrier()

        pltpu.sync_copy(agg_sh.at[pl.ds(s * _RPT, _RPT)],
                        out_h.at[c, pl.ds(s * _RPT, _RPT)])

    return k(p, src, dst)


def _tc_prep(feat, deg):
    def body(feat_r, deg_r, hn_r, ns_r, nd_r):
        sdeg = deg_r[0, 0:_N, 0:1]
        ddeg = deg_r[1, 0:_N, 0:1]
        ns = lax.rsqrt(jnp.maximum(sdeg, 1.0))
        nd = lax.rsqrt(jnp.maximum(ddeg, 1.0))
        ns_r[...] = ns
        nd_r[...] = nd
        hn_r[...] = feat_r[...] * ns

    return pl.pallas_call(
        body,
        out_shape=[
            jax.ShapeDtypeStruct((_N, _F), jnp.float32),
            jax.ShapeDtypeStruct((_N, 1), jnp.float32),
            jax.ShapeDtypeStruct((_N, 1), jnp.float32),
        ],
    )(feat, deg)


def _tc_layer(parts, h, ns, nd, w, l, g, be):
    def body(parts_r, h_r, ns_r, nd_r, w_r, l_r, g_r, be_r, out_r, hn_r):
        agg = (parts_r[0, 0:_N] + parts_r[1, 0:_N]) * nd_r[...]
        pre = jnp.dot(agg, w_r[...], preferred_element_type=jnp.float32)
        pre = pre + jnp.dot(h_r[...], l_r[...],
                            preferred_element_type=jnp.float32)
        m = jnp.mean(pre, axis=0, keepdims=True)
        d = pre - m
        v = jnp.mean(d * d, axis=0, keepdims=True)
        y = d * lax.rsqrt(v + 1e-5) * g_r[...] + be_r[...]
        y = jnp.maximum(y, 0.0)
        out_r[...] = y
        hn_r[...] = y * ns_r[...]

    return pl.pallas_call(
        body,
        out_shape=[
            jax.ShapeDtypeStruct((_N, _F), jnp.float32),
            jax.ShapeDtypeStruct((_N, _F), jnp.float32),
        ],
    )(parts, h, ns, nd, w, l, g, be)


def _tc_final(parts, h, nd, w, b, l):
    def body(parts_r, h_r, nd_r, w_r, b_r, l_r, out_r):
        agg = (parts_r[0, 0:_N] + parts_r[1, 0:_N]) * nd_r[...]
        out = jnp.dot(agg, w_r[...], preferred_element_type=jnp.float32)
        out = out + jnp.dot(h_r[...], l_r[...],
                            preferred_element_type=jnp.float32)
        out_r[...] = out + b_r[...]

    c = w.shape[1]
    return pl.pallas_call(
        body,
        out_shape=jax.ShapeDtypeStruct((_N, c), jnp.float32),
    )(parts, h, nd, w, b, l)


def kernel(feat, edge_index, W0, W1, W2, b2, L0, L1, L2, g0, be0, g1, be1):
    src = edge_index[0]
    dst = edge_index[1]

    deg = _sc_degrees(edge_index.reshape(-1))
    hn0, ns, nd = _tc_prep(feat, deg)

    parts0 = _sc_agg(hn0, src, dst)
    h1, hn1 = _tc_layer(parts0, feat, ns, nd, W0, L0,
                        g0.reshape(1, -1), be0.reshape(1, -1))

    parts1 = _sc_agg(hn1, src, dst)
    h2, hn2 = _tc_layer(parts1, h1, ns, nd, W1, L1,
                        g1.reshape(1, -1), be1.reshape(1, -1))

    parts2 = _sc_agg(hn2, src, dst)
    return _tc_final(parts2, h2, nd, W2, b2.reshape(1, -1), L2)


# speedup vs baseline: 9.6108x; 2.2661x over previous
import functools

import jax
import jax.numpy as jnp
from jax import lax
from jax.experimental import pallas as pl
from jax.experimental.pallas import tpu as pltpu
from jax.experimental.pallas import tpu_sc as plsc

_N = 10000
_E = 320000
_F = 128
_LANES = 16
_NC = 2
_NS = 16
_NW = _NC * _NS

_EPT = _E // _NW
_K = 40
_NCHUNK = _EPT // _K
_NBUF = 5
_NP = 10240
_RPT = _NP // _NS
_KD = 80


def _sc_mesh():
    return plsc.VectorSubcoreMesh(core_axis_name="c", subcore_axis_name="s")


def _sc_degrees(eidx):
    ept = _E // _NS
    nchunk = ept // _KD

    @functools.partial(
        pl.kernel,
        out_type=jax.ShapeDtypeStruct((_NC, _NP, _F), jnp.float32),
        mesh=_sc_mesh(),
        scratch_types=[
            pltpu.VMEM((ept,), jnp.int32),
            pltpu.VMEM((_KD, _F), jnp.float32),
            pltpu.SemaphoreType.DMA,
            pltpu.VMEM_SHARED((_NP, _F), jnp.float32),
        ],
    )
    def k(eidx_h, out_h, idx_v, ones_v, sem, cnt_sh):
        c = lax.axis_index("c")
        s = lax.axis_index("s")

        cp = pltpu.async_copy(eidx_h.at[pl.ds(c * _E + s * ept, ept)],
                              idx_v, sem)

        def fill(val):
            def f(i, _):
                ones_v[i // (_F // _LANES),
                       pl.ds((i % (_F // _LANES)) * _LANES, _LANES)] = (
                           jnp.full((_LANES,), val, jnp.float32))
                return 0
            lax.fori_loop(0, _KD * (_F // _LANES), f, 0)

        fill(0.0)
        for t in range(_RPT // _KD):
            pltpu.sync_copy(ones_v,
                            cnt_sh.at[pl.ds(s * _RPT + t * _KD, _KD)])
        fill(1.0)
        cp.wait()
        plsc.subcore_barrier()

        def body(j, _):
            pltpu.sync_copy(ones_v, cnt_sh.at[idx_v.at[pl.ds(j * _KD, _KD)]],
                            add=True)
            return 0

        lax.fori_loop(0, nchunk, body, 0)
        plsc.subcore_barrier()

        pltpu.sync_copy(cnt_sh.at[pl.ds(s * _RPT, _RPT)],
                        out_h.at[c, pl.ds(s * _RPT, _RPT)])

    return k(eidx)


def _sc_agg(p, src, dst):
    @functools.partial(
        pl.kernel,
        out_type=jax.ShapeDtypeStruct((_NC, _NP, _F), jnp.float32),
        mesh=_sc_mesh(),
        scratch_types=[
            pltpu.VMEM((_EPT,), jnp.int32),
            pltpu.VMEM((_EPT,), jnp.int32),
            [pltpu.VMEM((_K, _F), jnp.float32) for _ in range(_NBUF)],
            [pltpu.SemaphoreType.DMA for _ in range(_NBUF)],
            pltpu.VMEM_SHARED((_NP, _F), jnp.float32),
        ],
    )
    def k(p_h, src_h, dst_h, out_h, sidx, didx, rows, gsem, agg_sh):
        c = lax.axis_index("c")
        s = lax.axis_index("s")
        tbase = c * (_E // _NC) + s * _EPT

        cps = pltpu.async_copy(src_h.at[pl.ds(tbase, _EPT)], sidx, gsem[0])
        cpd = pltpu.async_copy(dst_h.at[pl.ds(tbase, _EPT)], didx, gsem[1])

        def zrow(i, _):
            rows[0][i // (_F // _LANES),
                    pl.ds((i % (_F // _LANES)) * _LANES, _LANES)] = (
                        jnp.zeros((_LANES,), jnp.float32))
            return 0

        lax.fori_loop(0, _K * (_F // _LANES), zrow, 0)
        for t in range(_RPT // _K):
            pltpu.sync_copy(rows[0],
                            agg_sh.at[pl.ds(s * _RPT + t * _K, _K)])
        cps.wait()
        cpd.wait()
        plsc.subcore_barrier()

        for b in range(_NBUF):
            pltpu.async_copy(p_h.at[sidx.at[pl.ds(b * _K, _K)]], rows[b],
                             gsem[b])

        def outer(o, _):
            for b in range(_NBUF):
                j = o * _NBUF + b
                pltpu.make_async_copy(p_h.at[sidx.at[pl.ds(0, _K)]],
                                      rows[b], gsem[b]).wait()
                pltpu.sync_copy(rows[b],
                                agg_sh.at[didx.at[pl.ds(j * _K, _K)]],
                                add=True)

                @pl.when(o < _NCHUNK // _NBUF - 1)
                def _():
                    pltpu.async_copy(
                        p_h.at[sidx.at[pl.ds((j + _NBUF) * _K, _K)]],
                        rows[b], gsem[b])
            return 0

        lax.fori_loop(0, _NCHUNK // _NBUF, outer, 0)
        plsc.subcore_barrier()

        pltpu.sync_copy(agg_sh.at[pl.ds(s * _RPT, _RPT)],
                        out_h.at[c, pl.ds(s * _RPT, _RPT)])

    return k(p, src, dst)


def _tc_prep(feat, deg):
    def body(feat_r, deg_r, hn_r, ns_r, nd_r):
        sdeg = deg_r[0, 0:_N, 0:1]
        ddeg = deg_r[1, 0:_N, 0:1]
        ns = lax.rsqrt(jnp.maximum(sdeg, 1.0))
        nd = lax.rsqrt(jnp.maximum(ddeg, 1.0))
        ns_r[...] = ns
        nd_r[...] = nd
        hn_r[...] = feat_r[...] * ns

    return pl.pallas_call(
        body,
        out_shape=[
            jax.ShapeDtypeStruct((_N, _F), jnp.float32),
            jax.ShapeDtypeStruct((_N, 1), jnp.float32),
            jax.ShapeDtypeStruct((_N, 1), jnp.float32),
        ],
    )(feat, deg)


def _tc_layer(parts, h, ns, nd, w, l, g, be):
    def body(parts_r, h_r, ns_r, nd_r, w_r, l_r, g_r, be_r, out_r, hn_r):
        agg = (parts_r[0, 0:_N] + parts_r[1, 0:_N]) * nd_r[...]
        pre = jnp.dot(agg, w_r[...], preferred_element_type=jnp.float32)
        pre = pre + jnp.dot(h_r[...], l_r[...],
                            preferred_element_type=jnp.float32)
        m = jnp.mean(pre, axis=0, keepdims=True)
        d = pre - m
        v = jnp.mean(d * d, axis=0, keepdims=True)
        y = d * lax.rsqrt(v + 1e-5) * g_r[...] + be_r[...]
        y = jnp.maximum(y, 0.0)
        out_r[...] = y
        hn_r[...] = y * ns_r[...]

    return pl.pallas_call(
        body,
        out_shape=[
            jax.ShapeDtypeStruct((_N, _F), jnp.float32),
            jax.ShapeDtypeStruct((_N, _F), jnp.float32),
        ],
    )(parts, h, ns, nd, w, l, g, be)


def _tc_final(parts, h, nd, w, b, l):
    def body(parts_r, h_r, nd_r, w_r, b_r, l_r, out_r):
        agg = (parts_r[0, 0:_N] + parts_r[1, 0:_N]) * nd_r[...]
        out = jnp.dot(agg, w_r[...], preferred_element_type=jnp.float32)
        out = out + jnp.dot(h_r[...], l_r[...],
                            preferred_element_type=jnp.float32)
        out_r[...] = out + b_r[...]

    c = w.shape[1]
    return pl.pallas_call(
        body,
        out_shape=jax.ShapeDtypeStruct((_N, c), jnp.float32),
    )(parts, h, nd, w, b, l)


def kernel(feat, edge_index, W0, W1, W2, b2, L0, L1, L2, g0, be0, g1, be1):
    src = edge_index[0]
    dst = edge_index[1]

    deg = _sc_degrees(edge_index.reshape(-1))
    hn0, ns, nd = _tc_prep(feat, deg)

    parts0 = _sc_agg(hn0, src, dst)
    h1, hn1 = _tc_layer(parts0, feat, ns, nd, W0, L0,
                        g0.reshape(1, -1), be0.reshape(1, -1))

    parts1 = _sc_agg(hn1, src, dst)
    h2, hn2 = _tc_layer(parts1, h1, ns, nd, W1, L1,
                        g1.reshape(1, -1), be1.reshape(1, -1))

    parts2 = _sc_agg(hn2, src, dst)
    return _tc_final(parts2, h2, nd, W2, b2.reshape(1, -1), L2)
